# baseline (speedup 1.0000x reference)
import jax
import jax.numpy as jnp
import numpy as np
from jax.experimental import pallas as pl
from jax.experimental.pallas import tpu as pltpu

N = 4096
D_IN = 256
H = 128
C = 32
K = 4
LAM_HAT = 0.9
A_SCAD = 3.7
EPS = 1e-8

BT = 512
NT = N // BT
NPAIRS = NT * (NT + 1) // 2
BP = 256


def _prep_kernel(A_ref, F_ref, W1_ref, b1_ref, W2_ref, b2_ref,
                 F0_ref, Dd_ref, Dsq_ref, dinv_ref):
    a = A_ref[...]
    dd = jnp.sum(a, axis=1, keepdims=True) + 1.0
    Dd_ref[...] = dd
    dsq = jnp.sqrt(dd)
    Dsq_ref[...] = dsq
    dinv_ref[...] = 1.0 / dsq
    h = jnp.maximum(
        jnp.dot(F_ref[...].astype(jnp.bfloat16),
                W1_ref[...].astype(jnp.bfloat16),
                preferred_element_type=jnp.float32) + b1_ref[...], 0.0)
    F0_ref[...] = (jnp.dot(h.astype(jnp.bfloat16),
                           W2_ref[...].astype(jnp.bfloat16),
                           preferred_element_type=jnp.float32) + b2_ref[...])


def _iter_kernel(ti_ref, tj_ref, lam_ref, A1_ref, A2_ref, Fc_ref, dinv_ref,
                 Dsq_ref, Dd_ref, F0_ref, out_ref,
                 P_acc, S_acc, XG_s, FB_s, SQ_s, SQT_s, DIT_s):
    p = pl.program_id(0)
    ti = ti_ref[p]
    tj = tj_ref[p]
    lam_k = lam_ref[0]
    lam = 1.0 / LAM_HAT - 1.0
    alam = A_SCAD * lam_k
    inv_c = 1.0 / (2.0 * (A_SCAD - 1.0) * lam_k)

    @pl.when(p == 0)
    def _():
        xn = Fc_ref[...] / Dsq_ref[...]
        sq = jnp.sum(xn * xn, axis=1, keepdims=True)
        XG_s[...] = xn.astype(jnp.bfloat16)
        FB_s[...] = Fc_ref[...].astype(jnp.bfloat16)
        SQ_s[...] = sq
        SQT_s[...] = sq.T
        DIT_s[...] = dinv_ref[...].T
        P_acc[...] = jnp.zeros_like(P_acc)
        S_acc[...] = jnp.zeros_like(S_acc)

    @pl.when(p < NPAIRS)
    def _():
        sqi = SQ_s[pl.ds(ti * BT, BT), :]
        sqj = SQT_s[:, pl.ds(tj * BT, BT)]
        g = jax.lax.dot_general(XG_s[pl.ds(ti * BT, BT), :],
                                XG_s[pl.ds(tj * BT, BT), :],
                                (((1,), (1,)), ((), ())),
                                preferred_element_type=jnp.float32)
        z = jnp.maximum(sqi + sqj - 2.0 * g, 0.0)
        y = jnp.sqrt(z)
        d = 1.0 / jnp.maximum(y, EPS)
        t = jnp.maximum(jnp.minimum(alam * inv_c - y * inv_c, 0.5), 0.0)
        w = t * d

        di_i = dinv_ref[pl.ds(ti * BT, BT), :]
        di_j = dinv_ref[pl.ds(tj * BT, BT), :]
        dit_i = DIT_s[:, pl.ds(ti * BT, BT)]
        dit_j = DIT_s[:, pl.ds(tj * BT, BT)]
        fb_i = FB_s[pl.ds(ti * BT, BT), :]
        fb_j = FB_s[pl.ds(tj * BT, BT), :]

        @pl.when(ti == tj)
        def _():
            row = jax.lax.broadcasted_iota(jnp.int32, (BT, BT), 0)
            col = jax.lax.broadcasted_iota(jnp.int32, (BT, BT), 1)
            wd = jnp.where(row == col, 0.0, w)
            a1 = A1_ref[...]
            S_acc[pl.ds(ti * BT, BT), :] += jnp.sum(
                wd * a1, axis=1, keepdims=True)
            m1 = (wd * (a1 * di_i * dit_j)).astype(jnp.bfloat16)
            P_acc[pl.ds(ti * BT, BT), :] += jax.lax.dot_general(
                m1, fb_j, (((1,), (0,)), ((), ())),
                preferred_element_type=jnp.float32)

        @pl.when(ti != tj)
        def _():
            a1 = A1_ref[...]
            S_acc[pl.ds(ti * BT, BT), :] += jnp.sum(
                w * a1, axis=1, keepdims=True)
            m1 = (w * (a1 * di_i * dit_j)).astype(jnp.bfloat16)
            P_acc[pl.ds(ti * BT, BT), :] += jax.lax.dot_general(
                m1, fb_j, (((1,), (0,)), ((), ())),
                preferred_element_type=jnp.float32)
            wt = w.T
            a2 = A2_ref[...]
            S_acc[pl.ds(tj * BT, BT), :] += jnp.sum(
                wt * a2, axis=1, keepdims=True)
            m2 = (wt * (a2 * di_j * dit_i)).astype(jnp.bfloat16)
            P_acc[pl.ds(tj * BT, BT), :] += jax.lax.dot_general(
                m2, fb_i, (((1,), (0,)), ((), ())),
                preferred_element_type=jnp.float32)

    @pl.when(p == NPAIRS)
    def _():
        q = S_acc[...] / Dd_ref[...] + lam
        out_ref[...] = P_acc[...] / q + (lam * F0_ref[...]) / q


def _prep_call(A, F, W1, b1, W2, b2):
    return pl.pallas_call(
        _prep_kernel,
        grid=(N // BP,),
        in_specs=[
            pl.BlockSpec((BP, N), lambda i: (i, 0)),
            pl.BlockSpec((BP, D_IN), lambda i: (i, 0)),
            pl.BlockSpec((D_IN, H), lambda i: (0, 0)),
            pl.BlockSpec((1, H), lambda i: (0, 0)),
            pl.BlockSpec((H, C), lambda i: (0, 0)),
            pl.BlockSpec((1, C), lambda i: (0, 0)),
        ],
        out_specs=[
            pl.BlockSpec((BP, C), lambda i: (i, 0)),
            pl.BlockSpec((BP, 1), lambda i: (i, 0)),
            pl.BlockSpec((BP, 1), lambda i: (i, 0)),
            pl.BlockSpec((BP, 1), lambda i: (i, 0)),
        ],
        out_shape=[
            jax.ShapeDtypeStruct((N, C), jnp.float32),
            jax.ShapeDtypeStruct((N, 1), jnp.float32),
            jax.ShapeDtypeStruct((N, 1), jnp.float32),
            jax.ShapeDtypeStruct((N, 1), jnp.float32),
        ],
        compiler_params=pltpu.CompilerParams(
            dimension_semantics=("arbitrary",)),
    )(A, F, W1, b1, W2, b2)


_TI_LIST = []
_TJ_LIST = []
for _a in range(NT):
    for _b in range(_a, NT):
        _TI_LIST.append(_a)
        _TJ_LIST.append(_b)
_TI_LIST.append(0)
_TJ_LIST.append(0)
_TI_ARR = np.asarray(_TI_LIST, np.int32)
_TJ_ARR = np.asarray(_TJ_LIST, np.int32)


def _iter_call(lam_k, A, Fc, dinv, Dsq, Dd, F0):
    grid_spec = pltpu.PrefetchScalarGridSpec(
        num_scalar_prefetch=3,
        grid=(NPAIRS + 1,),
        in_specs=[
            pl.BlockSpec((BT, BT), lambda p, ti, tj, lam: (ti[p], tj[p])),
            pl.BlockSpec((BT, BT), lambda p, ti, tj, lam: (tj[p], ti[p])),
            pl.BlockSpec((N, C), lambda p, ti, tj, lam: (0, 0)),
            pl.BlockSpec((N, 1), lambda p, ti, tj, lam: (0, 0)),
            pl.BlockSpec((N, 1), lambda p, ti, tj, lam: (0, 0)),
            pl.BlockSpec((N, 1), lambda p, ti, tj, lam: (0, 0)),
            pl.BlockSpec((N, C), lambda p, ti, tj, lam: (0, 0)),
        ],
        out_specs=pl.BlockSpec((N, C), lambda p, ti, tj, lam: (0, 0)),
        scratch_shapes=[
            pltpu.VMEM((N, C), jnp.float32),
            pltpu.VMEM((N, 1), jnp.float32),
            pltpu.VMEM((N, C), jnp.bfloat16),
            pltpu.VMEM((N, C), jnp.bfloat16),
            pltpu.VMEM((N, 1), jnp.float32),
            pltpu.VMEM((1, N), jnp.float32),
            pltpu.VMEM((1, N), jnp.float32),
        ],
    )
    return pl.pallas_call(
        _iter_kernel,
        grid_spec=grid_spec,
        out_shape=jax.ShapeDtypeStruct((N, C), jnp.float32),
        compiler_params=pltpu.CompilerParams(
            dimension_semantics=("arbitrary",)),
    )(jnp.asarray(_TI_ARR), jnp.asarray(_TJ_ARR), lam_k,
      A, A, Fc, dinv, Dsq, Dd, F0)


def kernel(A, F, W1, b1, W2, b2, log_lams):
    F0, Dd, Dsq, dinv = _prep_call(
        A, F, W1, b1.reshape(1, H), W2, b2.reshape(1, C))
    lams = jnp.exp(log_lams)
    Fc = F0
    for k in range(K):
        Fc = _iter_call(lams[k].reshape(1), A, Fc, dinv, Dsq, Dd, F0)
    return Fc
